# baseline (speedup 1.0000x reference)
import jax
import jax.numpy as jnp
from jax import lax
from jax.experimental import pallas as pl
from jax.experimental.pallas import tpu as pltpu
from jax.experimental.pallas import tpu_sc as plsc

H = 128
NH = 8
HD = H // NH
NT = 1024
SCALE = HD ** (-0.5)
NC = 100000
BC = 2000
NB = 50

CH = 128
NFULL = NC // CH
TAIL = NC - NFULL * CH
NW = 32

_SC_MESH = plsc.VectorSubcoreMesh(
    core_axis_name="c", subcore_axis_name="s", num_cores=2, num_subcores=16)


def _wid():
    return lax.axis_index("s") * 2 + lax.axis_index("c")


def _nchunks(w):
    return 24 + jnp.where(w < NFULL % NW, 1, 0)


def _sc_gather_body(table_hbm, idx_hbm, out_hbm,
                    idx_v, rows_v, idx_t, rows_t, sem):
    w = _wid()

    def body(k, carry):
        off = (k * NW + w) * CH
        pltpu.sync_copy(idx_hbm.at[pl.ds(off, CH)], idx_v)
        pltpu.async_copy(table_hbm.at[idx_v], rows_v, sem).wait()
        pltpu.sync_copy(rows_v, out_hbm.at[pl.ds(off, CH)])
        return carry

    lax.fori_loop(0, _nchunks(w), body, 0)

    @pl.when(w == NW - 1)
    def _():
        off = NFULL * CH
        pltpu.sync_copy(idx_hbm.at[pl.ds(off, TAIL)], idx_t)
        pltpu.async_copy(table_hbm.at[idx_t], rows_t, sem).wait()
        pltpu.sync_copy(rows_t, out_hbm.at[pl.ds(off, TAIL)])


def _sc_gather(table, idx):
    return pl.kernel(
        _sc_gather_body,
        out_type=jax.ShapeDtypeStruct((NC, H), jnp.float32),
        mesh=_SC_MESH,
        scratch_types=[
            pltpu.VMEM((CH,), jnp.int32),
            pltpu.VMEM((CH, H), jnp.float32),
            pltpu.VMEM((TAIL,), jnp.int32),
            pltpu.VMEM((TAIL, H), jnp.float32),
            pltpu.SemaphoreType.DMA,
        ],
    )(table, idx)


SCH = 80
SNCH = NC // SCH


def _sc_scatter_body(exv_hbm, idx_hbm, zatt_hbm, att_out,
                     idx_v, rows_v, att_sh):
    c = lax.axis_index("c")
    s = lax.axis_index("s")
    w = _wid()

    @pl.when(s == 0)
    def _():
        pltpu.sync_copy(zatt_hbm, att_sh)

    plsc.subcore_barrier()

    nk = 39 + jnp.where(w < SNCH % NW, 1, 0)

    def body(k, carry):
        off = (k * NW + w) * SCH
        pltpu.sync_copy(idx_hbm.at[pl.ds(off, SCH)], idx_v.at[0])
        pltpu.sync_copy(exv_hbm.at[pl.ds(off, SCH)], rows_v)
        pltpu.sync_copy(rows_v, att_sh.at[idx_v.at[0]], add=True)
        return carry

    lax.fori_loop(0, nk, body, 0)

    plsc.subcore_barrier()

    @pl.when(s == 0)
    def _():
        pltpu.sync_copy(att_sh, att_out.at[c])


def _sc_scatter(exv, idx):
    zatt = jnp.zeros((NT, H), jnp.float32)
    return pl.kernel(
        _sc_scatter_body,
        out_type=jax.ShapeDtypeStruct((2, NT, H), jnp.float32),
        mesh=_SC_MESH,
        scratch_types=[
            pltpu.VMEM((1, SCH), jnp.int32),
            pltpu.VMEM((SCH, H), jnp.float32),
            pltpu.VMEM_SHARED((NT, H), jnp.float32),
        ],
    )(exv, idx, zatt)


def _ln(x, g, b):
    mu = jnp.mean(x, axis=-1, keepdims=True)
    var = jnp.mean((x - mu) ** 2, axis=-1, keepdims=True)
    return (x - mu) * jax.lax.rsqrt(var + 1e-5) * g + b


def _head_expand():
    r = jax.lax.broadcasted_iota(jnp.int32, (NH, H), 0)
    c = jax.lax.broadcasted_iota(jnp.int32, (NH, H), 1)
    return (r == c // HD).astype(jnp.float32)


def _q_kernel(tis_ref, wqT_ref, bq_ref, q_ref):
    q_ref[...] = (
        jnp.dot(tis_ref[...], wqT_ref[...], preferred_element_type=jnp.float32)
        + bq_ref[...]
    )


def _score_kernel(cell_ref, qg_ref, wkT_ref, bk_ref, s_ref, m_ref, m_scr):
    i = pl.program_id(0)

    @pl.when(i == 0)
    def _():
        m_scr[0, 0] = -jnp.inf

    K = (jnp.dot(cell_ref[...], wkT_ref[...],
                 preferred_element_type=jnp.float32) + bk_ref[...])
    s = jnp.dot(qg_ref[...] * K, _head_expand().T,
                preferred_element_type=jnp.float32) * SCALE
    s_ref[...] = s
    m_scr[0, 0] = jnp.maximum(m_scr[0, 0], jnp.max(s))

    @pl.when(i == NB - 1)
    def _():
        m_ref[...] = jnp.full((1, 1), m_scr[0, 0], jnp.float32)


def _exv_kernel(cell_ref, s_ref, m_ref, lab_ref, wvT_ref, bv_ref,
                exv_ref, dn_ref, dn_scr):
    i = pl.program_id(0)

    @pl.when(i == 0)
    def _():
        dn_scr[...] = jnp.zeros_like(dn_scr)

    V = (jnp.dot(cell_ref[...], wvT_ref[...],
                 preferred_element_type=jnp.float32) + bv_ref[...])
    ex = jnp.exp(s_ref[...] - m_ref[...])
    exR = jnp.dot(ex, _head_expand(),
                  preferred_element_type=jnp.float32)
    exv_ref[...] = V * exR
    sel = (jax.lax.broadcasted_iota(jnp.int32, (NH, 16), 0)
           == jax.lax.broadcasted_iota(jnp.int32, (NH, 16), 1)
           ).astype(jnp.float32)
    col8 = (jax.lax.broadcasted_iota(jnp.int32, (BC, 16), 1) == 8
            ).astype(jnp.float32)
    dnr = jnp.dot(ex, sel, preferred_element_type=jnp.float32) + col8
    lab = lab_ref[0]
    tid = jax.lax.broadcasted_iota(jnp.int32, (NT, 1), 0)
    ohT = (tid == lab).astype(jnp.bfloat16)
    dn_scr[...] += jnp.dot(ohT, dnr.astype(jnp.bfloat16),
                           preferred_element_type=jnp.float32)

    @pl.when(i == NB - 1)
    def _():
        dn_ref[...] = dn_scr[...]


def _tissue_kernel(att2_ref, dn_ref, tis_ref, woT_ref, bo_ref,
                   tdwvT_ref, tdbv_ref, tdwoT_ref, tdbo_ref,
                   g_ref, b_ref, td_ref, tout_ref):
    dn = dn_ref[...]
    att_raw = att2_ref[0] + att2_ref[1]
    counts = dn[:, 8:9]
    mask = counts > 0.5
    denom = dn[:, :NH]
    denom = jnp.where(denom == 0.0, 1.0, denom)
    rep = jnp.dot(1.0 / denom, _head_expand(),
                  preferred_element_type=jnp.float32)
    att = att_raw * rep
    att_o = (jnp.dot(att, woT_ref[...],
                     preferred_element_type=jnp.float32) + bo_ref[...])
    tis = tis_ref[...]
    t_upd = jnp.where(mask, att_o, tis)
    td_v = (jnp.dot(t_upd, tdwvT_ref[...],
                    preferred_element_type=jnp.float32) + tdbv_ref[...])
    td_ref[...] = (jnp.dot(td_v, tdwoT_ref[...],
                           preferred_element_type=jnp.float32) + tdbo_ref[...])
    tout_ref[...] = _ln(tis + t_upd, g_ref[...], b_ref[...])


def _cell_kernel(cell_ref, gth_ref, g_ref, b_ref, out_ref):
    out_ref[...] = _ln(cell_ref[...] + gth_ref[...], g_ref[...], b_ref[...])


def _full(shape):
    return pl.BlockSpec(shape, lambda i: tuple(0 for _ in shape))


def kernel(cell_features, tissue_features, cluster_labels, tissue_batch,
           bu_Wq, bu_bq, bu_Wk, bu_bk, bu_Wv, bu_bv, bu_Wo, bu_bo,
           td_Wq, td_bq, td_Wk, td_bk, td_Wv, td_bv, td_Wo, td_bo,
           cell_ln_g, cell_ln_b, tissue_ln_g, tissue_ln_b):
    Q = pl.pallas_call(
        _q_kernel,
        out_shape=jax.ShapeDtypeStruct((NT, H), jnp.float32),
    )(tissue_features, bu_Wq.T, bu_bq.reshape(1, H))

    Qg = _sc_gather(Q, cluster_labels)

    scores, M = pl.pallas_call(
        _score_kernel,
        grid=(NB,),
        in_specs=[
            pl.BlockSpec((BC, H), lambda i: (i, 0)),
            pl.BlockSpec((BC, H), lambda i: (i, 0)),
            _full((H, H)),
            _full((1, H)),
        ],
        out_specs=[
            pl.BlockSpec((BC, NH), lambda i: (i, 0)),
            pl.BlockSpec((1, 1), lambda i: (0, 0)),
        ],
        out_shape=[
            jax.ShapeDtypeStruct((NC, NH), jnp.float32),
            jax.ShapeDtypeStruct((1, 1), jnp.float32),
        ],
        scratch_shapes=[pltpu.SMEM((1, 1), jnp.float32)],
    )(cell_features, Qg, bu_Wk.T, bu_bk.reshape(1, H))

    exv, dn = pl.pallas_call(
        _exv_kernel,
        grid=(NB,),
        in_specs=[
            pl.BlockSpec((BC, H), lambda i: (i, 0)),
            pl.BlockSpec((BC, NH), lambda i: (i, 0)),
            _full((1, 1)),
            pl.BlockSpec((1, 1, BC), lambda i: (i, 0, 0)),
            _full((H, H)),
            _full((1, H)),
        ],
        out_specs=[
            pl.BlockSpec((BC, H), lambda i: (i, 0)),
            pl.BlockSpec((NT, 16), lambda i: (0, 0)),
        ],
        out_shape=[
            jax.ShapeDtypeStruct((NC, H), jnp.float32),
            jax.ShapeDtypeStruct((NT, 16), jnp.float32),
        ],
        scratch_shapes=[pltpu.VMEM((NT, 16), jnp.float32)],
    )(cell_features, scores, M, cluster_labels.reshape(NB, 1, BC),
      bu_Wv.T, bu_bv.reshape(1, H))

    att2 = _sc_scatter(exv, cluster_labels)

    td_out, tissue_out = pl.pallas_call(
        _tissue_kernel,
        out_shape=[
            jax.ShapeDtypeStruct((NT, H), jnp.float32),
            jax.ShapeDtypeStruct((NT, H), jnp.float32),
        ],
    )(att2, dn, tissue_features, bu_Wo.T, bu_bo.reshape(1, H),
      td_Wv.T, td_bv.reshape(1, H), td_Wo.T, td_bo.reshape(1, H),
      tissue_ln_g.reshape(1, H), tissue_ln_b.reshape(1, H))

    G = _sc_gather(td_out, cluster_labels)

    cell_out = pl.pallas_call(
        _cell_kernel,
        grid=(NB,),
        in_specs=[
            pl.BlockSpec((BC, H), lambda i: (i, 0)),
            pl.BlockSpec((BC, H), lambda i: (i, 0)),
            _full((1, H)),
            _full((1, H)),
        ],
        out_specs=pl.BlockSpec((BC, H), lambda i: (i, 0)),
        out_shape=jax.ShapeDtypeStruct((NC, H), jnp.float32),
    )(cell_features, G,
      cell_ln_g.reshape(1, H), cell_ln_b.reshape(1, H))

    return cell_out, tissue_out
